# initial kernel scaffold (unmeasured)
import jax
import jax.numpy as jnp
from jax import lax
from jax.experimental import pallas as pl
from jax.experimental.pallas import tpu as pltpu


def kernel(
    x,
):
    def body(*refs):
        pass

    out_shape = jax.ShapeDtypeStruct(..., jnp.float32)
    return pl.pallas_call(body, out_shape=out_shape)(...)


# baseline (device time: 1098326 ns/iter reference)
import jax
import jax.numpy as jnp
from jax import lax
from jax.experimental import pallas as pl
from jax.experimental.pallas import tpu as pltpu

N_Y = 2


def kernel(x):
    m, n = x.shape
    xb = x.astype(jnp.bfloat16)

    def body(x_hbm, out_hbm, local_sem, send_sem, recv_sem):
        my_x = lax.axis_index("x")
        my_y = lax.axis_index("y")
        peer = (my_x, 1 - my_y)

        barrier_sem = pltpu.get_barrier_semaphore()
        pl.semaphore_signal(
            barrier_sem, inc=1, device_id=peer,
            device_id_type=pl.DeviceIdType.MESH,
        )
        pl.semaphore_wait(barrier_sem, 1)

        local = pltpu.make_async_copy(
            x_hbm, out_hbm.at[pl.ds(my_y * m, m)], local_sem
        )
        local.start()

        rdma = pltpu.make_async_remote_copy(
            src_ref=x_hbm,
            dst_ref=out_hbm.at[pl.ds(my_y * m, m)],
            send_sem=send_sem,
            recv_sem=recv_sem,
            device_id=peer,
            device_id_type=pl.DeviceIdType.MESH,
        )
        rdma.start()

        local.wait()
        rdma.wait()

    return pl.pallas_call(
        body,
        out_shape=jax.ShapeDtypeStruct((N_Y * m, n), jnp.bfloat16),
        in_specs=[pl.BlockSpec(memory_space=pl.ANY)],
        out_specs=pl.BlockSpec(memory_space=pl.ANY),
        scratch_shapes=[
            pltpu.SemaphoreType.DMA,
            pltpu.SemaphoreType.DMA,
            pltpu.SemaphoreType.DMA,
        ],
        compiler_params=pltpu.CompilerParams(collective_id=0),
    )(xb)


# device time: 449562 ns/iter; 2.4431x vs baseline; 2.4431x over previous
import jax
import jax.numpy as jnp
from jax import lax
from jax.experimental import pallas as pl
from jax.experimental.pallas import tpu as pltpu

N_Y = 2


def kernel(x):
    m, n = x.shape
    xb = x.astype(jnp.bfloat16)

    def body(x_hbm, out_hbm, local_sem, send_sem, recv_sem):
        my_x = lax.axis_index("x")
        my_y = lax.axis_index("y")
        peer = (my_x, 1 - my_y)

        barrier_sem = pltpu.get_barrier_semaphore()
        pl.semaphore_signal(
            barrier_sem, inc=1, device_id=peer,
            device_id_type=pl.DeviceIdType.MESH,
        )
        pl.semaphore_wait(barrier_sem, 1)

        local = pltpu.make_async_copy(
            x_hbm, out_hbm.at[pl.ds(my_y * m, m)], local_sem
        )
        local.start()

        rdma = pltpu.make_async_remote_copy(
            src_ref=x_hbm,
            dst_ref=out_hbm.at[pl.ds(my_y * m, m)],
            send_sem=send_sem,
            recv_sem=recv_sem,
            device_id=peer,
            device_id_type=pl.DeviceIdType.MESH,
        )
        rdma.start()

        local.wait()
        rdma.wait()

    return pl.pallas_call(
        body,
        out_shape=jax.ShapeDtypeStruct((N_Y * m, n), jnp.bfloat16),
        in_specs=[pl.BlockSpec(memory_space=pltpu.MemorySpace.VMEM)],
        out_specs=pl.BlockSpec(memory_space=pl.ANY),
        scratch_shapes=[
            pltpu.SemaphoreType.DMA,
            pltpu.SemaphoreType.DMA,
            pltpu.SemaphoreType.DMA,
        ],
        compiler_params=pltpu.CompilerParams(collective_id=0),
    )(xb)
